# Initial kernel scaffold; baseline (speedup 1.0000x reference)
import jax
import jax.numpy as jnp
from jax.experimental import pallas as pl


def kernel(x, backbone_w, backbone_b, cls_w, cls_b):
    raise NotImplementedError("write your pallas kernel here")


# speedup vs baseline: 4.7269x; 4.7269x over previous
import numpy as np
import jax
import jax.numpy as jnp
from jax.experimental import pallas as pl
from jax.experimental.pallas import tpu as pltpu


def _bilinear_matrix(out_size, in_size):
    scale = in_size / out_size
    idx = np.arange(out_size)
    src = (idx + 0.5) * scale - 0.5
    src = np.clip(src, 0.0, None)
    i0 = np.minimum(np.floor(src).astype(np.int64), in_size - 1)
    i1 = np.minimum(i0 + 1, in_size - 1)
    w1 = (src - i0).astype(np.float32)
    w0 = (1.0 - w1).astype(np.float32)
    A = np.zeros((out_size, in_size), dtype=np.float32)
    A[idx, i0] += w0
    A[idx, i1] += w1
    return A


def _fused_kernel(p_ref, w27_ref, bb_ref, wc_ref, bc_ref, ah_ref, awt_ref,
                  out_ref):
    _, NCLS, H, W = out_ref.shape
    Ho, K27, Wo = p_ref.shape[1:]
    OC = w27_ref.shape[0]

    P = p_ref[0]

    w_b = jnp.broadcast_to(w27_ref[...], (Ho, OC, K27))
    feat = jax.lax.dot_general(
        w_b, P, (((2,), (1,)), ((0,), (0,))),
        preferred_element_type=jnp.float32)
    feat = jnp.maximum(feat + bb_ref[...][None], 0.0).astype(jnp.bfloat16)

    wc_b = jnp.broadcast_to(wc_ref[...], (Ho, NCLS, OC))
    logits = jax.lax.dot_general(
        wc_b, feat, (((2,), (1,)), ((0,), (0,))),
        preferred_element_type=jnp.float32)
    logits = (logits + bc_ref[...][None]).astype(jnp.bfloat16)

    awt_b = jnp.broadcast_to(awt_ref[...], (Ho, Wo, W))
    t = jax.lax.dot_general(
        logits, awt_b, (((2,), (1,)), ((0,), (0,))),
        preferred_element_type=jnp.float32)
    t = t.astype(jnp.bfloat16)

    ah = ah_ref[...]
    for n in range(NCLS):
        y = jax.lax.dot_general(
            ah, t[:, n, :], (((1,), (0,)), ((), ())),
            preferred_element_type=jnp.float32)
        out_ref[0, n] = y


def kernel(x, backbone_w, backbone_b, cls_w, cls_b):
    N, C, H, W = x.shape
    OC = backbone_w.shape[0]
    NCLS = cls_w.shape[0]
    Ho = (H + 2 - 3) // 2 + 1
    Wo = (W + 2 - 3) // 2 + 1

    xpad = jnp.pad(x, ((0, 0), (0, 0), (1, 1), (1, 1)))
    taps = [xpad[:, :, i:i + 2 * Ho - 1:2, j:j + 2 * Wo - 1:2]
            for i in range(3) for j in range(3)]
    P = jnp.stack(taps, axis=2)
    P = P.reshape(N, 3 * C * 3, Ho, Wo).transpose(0, 2, 1, 3)
    P = P.astype(jnp.bfloat16)

    w27 = backbone_w.reshape(OC, 9 * C).astype(jnp.bfloat16)
    bb2 = backbone_b.reshape(OC, 1).astype(jnp.float32)
    wc2 = cls_w.reshape(NCLS, OC).astype(jnp.bfloat16)
    bc2 = cls_b.reshape(NCLS, 1).astype(jnp.float32)
    ah = jnp.asarray(_bilinear_matrix(H, Ho), jnp.bfloat16)
    awt = jnp.asarray(_bilinear_matrix(W, Wo).T, jnp.bfloat16)

    return pl.pallas_call(
        _fused_kernel,
        out_shape=jax.ShapeDtypeStruct((N, NCLS, H, W), jnp.float32),
        grid=(N,),
        in_specs=[
            pl.BlockSpec((1, Ho, 9 * C, Wo), lambda n: (n, 0, 0, 0)),
            pl.BlockSpec((OC, 9 * C), lambda n: (0, 0)),
            pl.BlockSpec((OC, 1), lambda n: (0, 0)),
            pl.BlockSpec((NCLS, OC), lambda n: (0, 0)),
            pl.BlockSpec((NCLS, 1), lambda n: (0, 0)),
            pl.BlockSpec((H, Ho), lambda n: (0, 0)),
            pl.BlockSpec((Wo, W), lambda n: (0, 0)),
        ],
        out_specs=pl.BlockSpec((1, NCLS, H, W), lambda n: (n, 0, 0, 0)),
        compiler_params=pltpu.CompilerParams(dimension_semantics=("parallel",)),
    )(P, w27, bb2, wc2, bc2, ah, awt)


# speedup vs baseline: 8.4660x; 1.7910x over previous
import numpy as np
import jax
import jax.numpy as jnp
from jax.experimental import pallas as pl
from jax.experimental.pallas import tpu as pltpu


def _bilinear_matrix(out_size, in_size):
    scale = in_size / out_size
    idx = np.arange(out_size)
    src = (idx + 0.5) * scale - 0.5
    src = np.clip(src, 0.0, None)
    i0 = np.minimum(np.floor(src).astype(np.int64), in_size - 1)
    i1 = np.minimum(i0 + 1, in_size - 1)
    w1 = (src - i0).astype(np.float32)
    w0 = (1.0 - w1).astype(np.float32)
    A = np.zeros((out_size, in_size), dtype=np.float32)
    A[idx, i0] += w0
    A[idx, i1] += w1
    return A


def _fused_kernel(p_ref, w27_ref, bb_ref, wc_ref, bc_ref, ah_ref, awt2_ref,
                  out_ref):
    _, NCLS, H, W = out_ref.shape
    Ho, K27, W2o = p_ref.shape[1:]
    OC = w27_ref.shape[0]

    P = p_ref[0]

    w_b = jnp.broadcast_to(w27_ref[...], (Ho, OC, K27))
    feat = jax.lax.dot_general(
        w_b, P, (((2,), (1,)), ((0,), (0,))),
        preferred_element_type=jnp.float32)
    feat = jnp.maximum(feat + bb_ref[...][None], 0.0).astype(jnp.bfloat16)

    wc_b = jnp.broadcast_to(wc_ref[...], (Ho, NCLS, OC))
    logits = jax.lax.dot_general(
        wc_b, feat, (((2,), (1,)), ((0,), (0,))),
        preferred_element_type=jnp.float32)
    logits = (logits + bc_ref[...][None]).astype(jnp.bfloat16)

    awt_b = jnp.broadcast_to(awt2_ref[...], (Ho, W2o, 2 * W))
    t = jax.lax.dot_general(
        logits, awt_b, (((2,), (1,)), ((0,), (0,))),
        preferred_element_type=jnp.float32)
    t = t.astype(jnp.bfloat16)

    ah = ah_ref[...]
    for n in range(NCLS):
        y = jax.lax.dot_general(
            ah, t[:, n, :], (((1,), (0,)), ((), ())),
            preferred_element_type=jnp.float32)
        out_ref[0, n] = y[:, :W]
        out_ref[1, n] = y[:, W:]


def kernel(x, backbone_w, backbone_b, cls_w, cls_b):
    N, C, H, W = x.shape
    OC = backbone_w.shape[0]
    NCLS = cls_w.shape[0]
    Ho = (H + 2 - 3) // 2 + 1
    Wo = (W + 2 - 3) // 2 + 1
    Hp, Wp = Ho + 1, Wo + 1

    xpad = jnp.pad(x, ((0, 0), (0, 0), (1, 2 * Hp - H - 1),
                       (1, 2 * Wp - W - 1)))
    ph = xpad.reshape(N, C, Hp, 2, Wp, 2)
    ph = ph.transpose(0, 3, 5, 1, 2, 4).reshape(N, 4 * C, Hp, Wp)
    ph = ph.astype(jnp.bfloat16)

    taps = []
    for c in range(C):
        for i in range(3):
            for j in range(3):
                q = (2 * (i % 2) + (j % 2)) * C + c
                a, b = i // 2, j // 2
                taps.append(ph[:, q, a:a + Ho, b:b + Wo])
    P = jnp.stack(taps, axis=1)
    P = P.reshape(N // 2, 2, 9 * C, Ho, Wo)
    P = P.transpose(0, 3, 2, 1, 4).reshape(N // 2, Ho, 9 * C, 2 * Wo)

    w27 = backbone_w.reshape(OC, 9 * C).astype(jnp.bfloat16)
    bb2 = backbone_b.reshape(OC, 1).astype(jnp.float32)
    wc2 = cls_w.reshape(NCLS, OC).astype(jnp.bfloat16)
    bc2 = cls_b.reshape(NCLS, 1).astype(jnp.float32)
    ah = jnp.asarray(_bilinear_matrix(H, Ho), jnp.bfloat16)
    awt = _bilinear_matrix(W, Wo).T
    awt2 = np.zeros((2 * Wo, 2 * W), np.float32)
    awt2[:Wo, :W] = awt
    awt2[Wo:, W:] = awt
    awt2 = jnp.asarray(awt2, jnp.bfloat16)

    out = pl.pallas_call(
        _fused_kernel,
        out_shape=jax.ShapeDtypeStruct((N, NCLS, H, W), jnp.float32),
        grid=(N // 2,),
        in_specs=[
            pl.BlockSpec((1, Ho, 9 * C, 2 * Wo), lambda n: (n, 0, 0, 0)),
            pl.BlockSpec((OC, 9 * C), lambda n: (0, 0)),
            pl.BlockSpec((OC, 1), lambda n: (0, 0)),
            pl.BlockSpec((NCLS, OC), lambda n: (0, 0)),
            pl.BlockSpec((NCLS, 1), lambda n: (0, 0)),
            pl.BlockSpec((H, Ho), lambda n: (0, 0)),
            pl.BlockSpec((2 * Wo, 2 * W), lambda n: (0, 0)),
        ],
        out_specs=pl.BlockSpec((2, NCLS, H, W), lambda n: (n, 0, 0, 0)),
        compiler_params=pltpu.CompilerParams(dimension_semantics=("parallel",)),
    )(P, w27, bb2, wc2, bc2, ah, awt2)
    return out
